# baseline (speedup 1.0000x reference)
import functools

import jax
import jax.numpy as jnp
from jax import lax
from jax.experimental import pallas as pl
from jax.experimental.pallas import tpu as pltpu
from jax.experimental.pallas import tpu_sc as plsc

_B, _H, _W, _C = 2, 384, 384, 96
_HW = _H * _W
_P = _B * _HW
_CP = 128
_N = 64
_NC, _NS = 2, 16
_NW = _NC * _NS
_CHUNKS = _P // _N
_CPW = _CHUNKS // _NW
_NB = _C // 16
_RB = 8
_NROW = _RB * _W


def _prep_body(trf_ref, idx_ref, w00_ref, w01_ref, w10_ref, w11_ref):
    b = pl.program_id(0)
    r = pl.program_id(1)
    t = trf_ref[0]
    dx = t[:, 0, :]
    dy = t[:, 1, :]
    gi = ((lax.broadcasted_iota(jnp.int32, (_RB, _W), 0)
           + r * _RB).astype(jnp.float32))
    gj = lax.broadcasted_iota(jnp.int32, (_RB, _W), 1).astype(jnp.float32)
    cx = jnp.minimum(jnp.maximum(gi + dx, 0.0), float(_H - 1))
    cy = jnp.minimum(jnp.maximum(gj + dy, 0.0), float(_W - 1))
    xb = jnp.minimum(cx.astype(jnp.int32), _H - 2)
    yb = jnp.minimum(cy.astype(jnp.int32), _W - 2)
    fx = cx - xb.astype(jnp.float32)
    fy = cy - yb.astype(jnp.float32)
    rb = xb * _W + yb + b * _HW
    nch = _NROW // _CP
    idx_ref[...] = rb.reshape(nch, _CP)
    wxa = 1.0 - fx
    wya = 1.0 - fy
    w00_ref[...] = (wxa * wya).reshape(nch, _CP)
    w01_ref[...] = (wxa * fy).reshape(nch, _CP)
    w10_ref[...] = (fx * wya).reshape(nch, _CP)
    w11_ref[...] = (fx * fy).reshape(nch, _CP)


_IDXROWS = _P // _CP


def _planar_spec():
    return pl.BlockSpec((_NROW // _CP, _CP),
                        lambda b, r: (b * (_H // _RB) + r, 0))


_prep = pl.pallas_call(
    _prep_body,
    grid=(_B, _H // _RB),
    in_specs=[pl.BlockSpec((1, _RB, 2, _W), lambda b, r: (b, r, 0, 0))],
    out_specs=[_planar_spec() for _ in range(5)],
    out_shape=[jax.ShapeDtypeStruct((_IDXROWS, _CP), jnp.int32)]
    + [jax.ShapeDtypeStruct((_IDXROWS, _CP), jnp.float32) for _ in range(4)],
)


def _pre_body(img_ref, tab_ref):
    x = img_ref[0]
    y = jnp.transpose(x, (0, 2, 1)).reshape(_NROW, _C)
    tab_ref[...] = jnp.concatenate(
        [y, jnp.zeros((_NROW, _CP - _C), jnp.float32)], axis=1)


_pre = pl.pallas_call(
    _pre_body,
    grid=(_B, _H // _RB),
    in_specs=[pl.BlockSpec((1, _RB, _C, _W), lambda b, r: (b, r, 0, 0))],
    out_specs=[pl.BlockSpec((_NROW, _CP),
                            lambda b, r: (b * (_H // _RB) + r, 0))],
    out_shape=[jax.ShapeDtypeStruct((_P, _CP), jnp.float32)],
)


def _post_body(tab_ref, img_ref):
    y = tab_ref[:, :_C]
    img_ref[0] = jnp.transpose(y.reshape(_RB, _W, _C), (0, 2, 1))


_post = pl.pallas_call(
    _post_body,
    grid=(_B, _H // _RB),
    in_specs=[pl.BlockSpec((_NROW, _CP),
                           lambda b, r: (b * (_H // _RB) + r, 0))],
    out_specs=[pl.BlockSpec((1, _RB, _C, _W), lambda b, r: (b, r, 0, 0))],
    out_shape=[jax.ShapeDtypeStruct((_B, _H, _C, _W), jnp.float32)],
)


def _warp_body(img_hbm, idxb_hbm, w00_hbm, w01_hbm, w10_hbm, w11_hbm,
               out_hbm, ib_v, idx_v, w_v, g_v, out_v, sem_in, sem_g, sem_out):
    wid = lax.axis_index("s") * _NC + lax.axis_index("c")
    w_hbms = (w00_hbm, w01_hbm, w10_hbm, w11_hbm)

    def chunk_of(i):
        return i * _NW + wid

    def in_copies(i, s):
        c = chunk_of(i)
        r2 = c // 2
        off = (c % 2) * _N
        cps = [pltpu.make_async_copy(
            idxb_hbm.at[r2, pl.ds(off, _N)], ib_v[s], sem_in[s])]
        for k in range(4):
            cps.append(pltpu.make_async_copy(
                w_hbms[k].at[r2, pl.ds(off, _N)], w_v[s].at[k], sem_in[s]))
        return cps

    def expand_idx(s):
        for g in range(_N // 16):
            sl = pl.ds(g * 16, 16)
            rv8 = ib_v[s][sl] * 8
            for c, delta in enumerate((0, 1, _W, _W + 1)):
                for blk in range(_NB):
                    j = blk * _N + g * 16
                    idx_v[s][c * 3 + j // 128, pl.ds(j % 128, 16)] = (
                        rv8 + (delta * 8 + blk))

    def gather_copies(i, s):
        cps = []
        for c in range(4):
            for q in range(3):
                cps.append(pltpu.make_async_copy(
                    img_hbm.at[idx_v[s].at[c * 3 + q]],
                    g_v[s][c].at[pl.ds(q * 128, 128)], sem_g[s]))
        return cps

    def out_copy(i):
        c = chunk_of(i)
        return pltpu.make_async_copy(
            out_v, out_hbm.at[pl.ds(c * _N, _N)], sem_out)

    def fire(copies):
        for cp in copies:
            cp.start()

    def drain(copies):
        for cp in copies:
            cp.wait()

    def combine(i, s):
        gs = g_v[s]

        def grp_body(gq, _):
            b16 = gq * 16
            w0g = w_v[s][0, pl.ds(b16, 16)]
            w1g = w_v[s][1, pl.ds(b16, 16)]
            w2g = w_v[s][2, pl.ds(b16, 16)]
            w3g = w_v[s][3, pl.ds(b16, 16)]
            for li in range(16):
                p = b16 + li
                w0 = _lane_bcast(w0g, li)
                w1 = _lane_bcast(w1g, li)
                w2 = _lane_bcast(w2g, li)
                w3 = _lane_bcast(w3g, li)
                for blk in range(_NB):
                    csl = pl.ds(blk * 16, 16)
                    row = blk * _N + p
                    out_v[p, csl] = (
                        w0 * gs[0][row, :] + w1 * gs[1][row, :]
                        + w2 * gs[2][row, :] + w3 * gs[3][row, :])
            return 0

        lax.fori_loop(0, _N // 16, grp_body, 0)

    fire(in_copies(0, 0))
    fire(in_copies(1, 1))
    drain(in_copies(0, 0))
    expand_idx(0)
    fire(gather_copies(0, 0))

    def pair_body(k, _):
        for s in (0, 1):
            i = k * 2 + s

            @pl.when(i < _CPW - 1)
            def _():
                drain(in_copies(i + 1, 1 - s))
                expand_idx(1 - s)
                fire(gather_copies(i + 1, 1 - s))

            drain(gather_copies(i, s))

            @pl.when(i >= 1)
            def _():
                out_copy(i - 1).wait()

            combine(i, s)
            out_copy(i).start()

            @pl.when(i < _CPW - 2)
            def _():
                fire(in_copies(i + 2, s))
        return 0

    lax.fori_loop(0, _CPW // 2, pair_body, 0)
    out_copy(_CPW - 1).wait()


_GATHER_DNUMS = lax.GatherDimensionNumbers(
    offset_dims=(), collapsed_slice_dims=(0,), start_index_map=(0,))


def _lane_bcast(vec, lane):
    lidx = jnp.full((16, 1), lane, jnp.int32)
    return lax.gather(vec, lidx, _GATHER_DNUMS, (1,),
                      mode=lax.GatherScatterMode.PROMISE_IN_BOUNDS)


_warp = functools.partial(
    pl.kernel,
    out_type=jax.ShapeDtypeStruct((_P, _CP), jnp.float32),
    mesh=plsc.VectorSubcoreMesh(core_axis_name="c", subcore_axis_name="s"),
    compiler_params=pltpu.CompilerParams(use_tc_tiling_on_sc=False),
    scratch_types=[
        [pltpu.VMEM((_N,), jnp.int32) for _ in range(2)],
        [pltpu.VMEM((12, 128), jnp.int32) for _ in range(2)],
        [pltpu.VMEM((4, _N), jnp.float32) for _ in range(2)],
        [[pltpu.VMEM((_NB * _N, 16), jnp.float32) for _ in range(4)]
         for _ in range(2)],
        pltpu.VMEM((_N, _CP), jnp.float32),
        [pltpu.SemaphoreType.DMA for _ in range(2)],
        [pltpu.SemaphoreType.DMA for _ in range(2)],
        pltpu.SemaphoreType.DMA,
    ],
)(_warp_body)


def kernel(img, trf):
    imgp, = _pre(jnp.transpose(img, (0, 1, 3, 2)))
    idxb, w00, w01, w10, w11 = _prep(jnp.transpose(trf, (0, 1, 3, 2)))
    outp = _warp(imgp.reshape(_P * 8, 16), idxb, w00, w01, w10, w11)
    outt, = _post(outp)
    return jnp.transpose(outt, (0, 1, 3, 2)), trf


# speedup vs baseline: 1.2943x; 1.2943x over previous
import functools

import jax
import jax.numpy as jnp
from jax import lax
from jax.experimental import pallas as pl
from jax.experimental.pallas import tpu as pltpu
from jax.experimental.pallas import tpu_sc as plsc

_B, _H, _W, _C = 2, 384, 384, 96
_HW = _H * _W
_P = _B * _HW
_CP = 128
_N = 64
_NC, _NS = 2, 16
_NW = _NC * _NS
_CHUNKS = _P // _N
_CPW = _CHUNKS // _NW
_NB = _C // 16
_RB = 8
_NROW = _RB * _W


def _prep_body(trf_ref, idx_ref, w00_ref, w01_ref, w10_ref, w11_ref):
    b = pl.program_id(0)
    r = pl.program_id(1)
    t = trf_ref[0]
    dx = t[:, 0, :]
    dy = t[:, 1, :]
    gi = ((lax.broadcasted_iota(jnp.int32, (_RB, _W), 0)
           + r * _RB).astype(jnp.float32))
    gj = lax.broadcasted_iota(jnp.int32, (_RB, _W), 1).astype(jnp.float32)
    cx = jnp.minimum(jnp.maximum(gi + dx, 0.0), float(_H - 1))
    cy = jnp.minimum(jnp.maximum(gj + dy, 0.0), float(_W - 1))
    xb = jnp.minimum(cx.astype(jnp.int32), _H - 2)
    yb = jnp.minimum(cy.astype(jnp.int32), _W - 2)
    fx = cx - xb.astype(jnp.float32)
    fy = cy - yb.astype(jnp.float32)
    rb = xb * _W + yb + b * _HW
    nch = _NROW // _CP
    idx_ref[...] = rb.reshape(nch, _CP)
    wxa = 1.0 - fx
    wya = 1.0 - fy
    w00_ref[...] = (wxa * wya).reshape(nch, _CP)
    w01_ref[...] = (wxa * fy).reshape(nch, _CP)
    w10_ref[...] = (fx * wya).reshape(nch, _CP)
    w11_ref[...] = (fx * fy).reshape(nch, _CP)


_IDXROWS = _P // _CP


def _planar_spec():
    return pl.BlockSpec((_NROW // _CP, _CP),
                        lambda b, r: (b * (_H // _RB) + r, 0))


_prep = pl.pallas_call(
    _prep_body,
    grid=(_B, _H // _RB),
    in_specs=[pl.BlockSpec((1, _RB, 2, _W), lambda b, r: (b, r, 0, 0))],
    out_specs=[_planar_spec() for _ in range(5)],
    out_shape=[jax.ShapeDtypeStruct((_IDXROWS, _CP), jnp.int32)]
    + [jax.ShapeDtypeStruct((_IDXROWS, _CP), jnp.float32) for _ in range(4)],
)


def _pre_body(img_ref, tab_ref):
    x = img_ref[0]
    y = jnp.transpose(x, (0, 2, 1)).reshape(_NROW, _C)
    tab_ref[...] = jnp.concatenate(
        [y, jnp.zeros((_NROW, _CP - _C), jnp.float32)], axis=1)


_pre = pl.pallas_call(
    _pre_body,
    grid=(_B, _H // _RB),
    in_specs=[pl.BlockSpec((1, _RB, _C, _W), lambda b, r: (b, r, 0, 0))],
    out_specs=[pl.BlockSpec((_NROW, _CP),
                            lambda b, r: (b * (_H // _RB) + r, 0))],
    out_shape=[jax.ShapeDtypeStruct((_P, _CP), jnp.float32)],
)


def _post_body(tab_ref, img_ref):
    y = tab_ref[:, :_C]
    img_ref[0] = jnp.transpose(y.reshape(_RB, _W, _C), (0, 2, 1))


_post = pl.pallas_call(
    _post_body,
    grid=(_B, _H // _RB),
    in_specs=[pl.BlockSpec((_NROW, _CP),
                           lambda b, r: (b * (_H // _RB) + r, 0))],
    out_specs=[pl.BlockSpec((1, _RB, _C, _W), lambda b, r: (b, r, 0, 0))],
    out_shape=[jax.ShapeDtypeStruct((_B, _H, _C, _W), jnp.float32)],
)


def _warp_body(img_hbm, idxb_hbm, w00_hbm, w01_hbm, w10_hbm, w11_hbm,
               out_hbm, ib_v, idx_v, w_v, g_v, out_v, sem_in, sem_g, sem_out):
    wid = lax.axis_index("s") * _NC + lax.axis_index("c")
    w_hbms = (w00_hbm, w01_hbm, w10_hbm, w11_hbm)

    def chunk_of(i):
        return i * _NW + wid

    def in_copies(i, s):
        c = chunk_of(i)
        r2 = c // 2
        off = (c % 2) * _N
        cps = [pltpu.make_async_copy(
            idxb_hbm.at[r2, pl.ds(off, _N)], ib_v[s], sem_in[s])]
        for k in range(4):
            cps.append(pltpu.make_async_copy(
                w_hbms[k].at[r2, pl.ds(off, _N)], w_v[s].at[k], sem_in[s]))
        return cps

    def expand_idx(s):
        for g in range(_N // 16):
            rv = ib_v[s][pl.ds(g * 16, 16)]
            for c, delta in enumerate((0, 1, _W, _W + 1)):
                j = c * _N + g * 16
                idx_v[s][j // 128, pl.ds(j % 128, 16)] = rv + delta

    def gather_copies(i, s):
        return tuple(
            pltpu.make_async_copy(
                img_hbm.at[idx_v[s].at[q]],
                g_v[s].at[pl.ds(q * 128, 128)], sem_g[s])
            for q in range(2))

    def out_copy(i):
        c = chunk_of(i)
        return pltpu.make_async_copy(
            out_v, out_hbm.at[pl.ds(c * _N, _N)], sem_out)

    def fire(copies):
        for cp in copies:
            cp.start()

    def drain(copies):
        for cp in copies:
            cp.wait()

    def combine(i, s):
        gs = g_v[s]

        def grp_body(gq, _):
            b16 = gq * 16
            w0g = w_v[s][0, pl.ds(b16, 16)]
            w1g = w_v[s][1, pl.ds(b16, 16)]
            w2g = w_v[s][2, pl.ds(b16, 16)]
            w3g = w_v[s][3, pl.ds(b16, 16)]
            for li in range(16):
                p = b16 + li
                w0 = _lane_bcast(w0g, li)
                w1 = _lane_bcast(w1g, li)
                w2 = _lane_bcast(w2g, li)
                w3 = _lane_bcast(w3g, li)
                for blk in range(_NB):
                    csl = pl.ds(blk * 16, 16)
                    out_v[p, csl] = (
                        w0 * gs[0 * _N + p, csl] + w1 * gs[1 * _N + p, csl]
                        + w2 * gs[2 * _N + p, csl] + w3 * gs[3 * _N + p, csl])
            return 0

        lax.fori_loop(0, _N // 16, grp_body, 0)

    fire(in_copies(0, 0))
    fire(in_copies(1, 1))
    drain(in_copies(0, 0))
    expand_idx(0)
    fire(gather_copies(0, 0))

    def pair_body(k, _):
        for s in (0, 1):
            i = k * 2 + s

            @pl.when(i < _CPW - 1)
            def _():
                drain(in_copies(i + 1, 1 - s))
                expand_idx(1 - s)
                fire(gather_copies(i + 1, 1 - s))

            drain(gather_copies(i, s))

            @pl.when(i >= 1)
            def _():
                out_copy(i - 1).wait()

            combine(i, s)
            out_copy(i).start()

            @pl.when(i < _CPW - 2)
            def _():
                fire(in_copies(i + 2, s))
        return 0

    lax.fori_loop(0, _CPW // 2, pair_body, 0)
    out_copy(_CPW - 1).wait()


_GATHER_DNUMS = lax.GatherDimensionNumbers(
    offset_dims=(), collapsed_slice_dims=(0,), start_index_map=(0,))


def _lane_bcast(vec, lane):
    lidx = jnp.full((16, 1), lane, jnp.int32)
    return lax.gather(vec, lidx, _GATHER_DNUMS, (1,),
                      mode=lax.GatherScatterMode.PROMISE_IN_BOUNDS)


_warp = functools.partial(
    pl.kernel,
    out_type=jax.ShapeDtypeStruct((_P, _CP), jnp.float32),
    mesh=plsc.VectorSubcoreMesh(core_axis_name="c", subcore_axis_name="s"),
    compiler_params=pltpu.CompilerParams(use_tc_tiling_on_sc=False),
    scratch_types=[
        [pltpu.VMEM((_N,), jnp.int32) for _ in range(2)],
        [pltpu.VMEM((2, 128), jnp.int32) for _ in range(2)],
        [pltpu.VMEM((4, _N), jnp.float32) for _ in range(2)],
        [pltpu.VMEM((4 * _N, _CP), jnp.float32)
         for _ in range(2)],
        pltpu.VMEM((_N, _CP), jnp.float32),
        [pltpu.SemaphoreType.DMA for _ in range(2)],
        [pltpu.SemaphoreType.DMA for _ in range(2)],
        pltpu.SemaphoreType.DMA,
    ],
)(_warp_body)


def kernel(img, trf):
    imgp, = _pre(jnp.transpose(img, (0, 1, 3, 2)))
    idxb, w00, w01, w10, w11 = _prep(jnp.transpose(trf, (0, 1, 3, 2)))
    outp = _warp(imgp, idxb, w00, w01, w10, w11)
    outt, = _post(outp)
    return jnp.transpose(outt, (0, 1, 3, 2)), trf


# speedup vs baseline: 1.3283x; 1.0262x over previous
import functools

import jax
import jax.numpy as jnp
from jax import lax
from jax.experimental import pallas as pl
from jax.experimental.pallas import tpu as pltpu
from jax.experimental.pallas import tpu_sc as plsc

_B, _H, _W, _C = 2, 384, 384, 96
_HW = _H * _W
_CP = 128
_N = 64
_NC, _NS = 2, 16
_NW = _NC * _NS
_CHUNKS = _HW // _N
_CPW = _CHUNKS // _NW
_NB = _C // 16
_RB = 8
_NROW = _RB * _W
_IDXROWS = _HW // _CP


def _prep_body(trf_ref, idx_ref, w00_ref, w01_ref, w10_ref, w11_ref):
    r = pl.program_id(0)
    t = trf_ref[...]
    dx = t[:, 0, :]
    dy = t[:, 1, :]
    gi = ((lax.broadcasted_iota(jnp.int32, (_RB, _W), 0)
           + r * _RB).astype(jnp.float32))
    gj = lax.broadcasted_iota(jnp.int32, (_RB, _W), 1).astype(jnp.float32)
    cx = jnp.minimum(jnp.maximum(gi + dx, 0.0), float(_H - 1))
    cy = jnp.minimum(jnp.maximum(gj + dy, 0.0), float(_W - 1))
    xb = jnp.minimum(cx.astype(jnp.int32), _H - 2)
    yb = jnp.minimum(cy.astype(jnp.int32), _W - 2)
    fx = cx - xb.astype(jnp.float32)
    fy = cy - yb.astype(jnp.float32)
    rb = xb * _W + yb
    nch = _NROW // _CP
    idx_ref[...] = rb.reshape(nch, _CP)
    wxa = 1.0 - fx
    wya = 1.0 - fy
    w00_ref[...] = (wxa * wya).reshape(nch, _CP)
    w01_ref[...] = (wxa * fy).reshape(nch, _CP)
    w10_ref[...] = (fx * wya).reshape(nch, _CP)
    w11_ref[...] = (fx * fy).reshape(nch, _CP)


def _planar_spec():
    return pl.BlockSpec((_NROW // _CP, _CP), lambda r: (r, 0))


_prep = pl.pallas_call(
    _prep_body,
    grid=(_H // _RB,),
    in_specs=[pl.BlockSpec((_RB, 2, _W), lambda r: (r, 0, 0))],
    out_specs=[_planar_spec() for _ in range(5)],
    out_shape=[jax.ShapeDtypeStruct((_IDXROWS, _CP), jnp.int32)]
    + [jax.ShapeDtypeStruct((_IDXROWS, _CP), jnp.float32) for _ in range(4)],
)


def _pre_body(img_ref, tab_ref):
    x = img_ref[...]
    y = jnp.transpose(x, (0, 2, 1)).reshape(_NROW, _C)
    tab_ref[...] = jnp.concatenate(
        [y, jnp.zeros((_NROW, _CP - _C), jnp.float32)], axis=1)


_pre = pl.pallas_call(
    _pre_body,
    grid=(_H // _RB,),
    in_specs=[pl.BlockSpec((_RB, _C, _W), lambda r: (r, 0, 0))],
    out_specs=[pl.BlockSpec((_NROW, _CP), lambda r: (r, 0))],
    out_shape=[jax.ShapeDtypeStruct((_HW, _CP), jnp.float32)],
)


def _post0_body(tab_ref, img_ref):
    y = tab_ref[:, :_C]
    img_ref[0] = jnp.transpose(y.reshape(_RB, _W, _C), (0, 2, 1))


_post0 = pl.pallas_call(
    _post0_body,
    grid=(_H // _RB,),
    in_specs=[pl.BlockSpec((_NROW, _CP), lambda r: (r, 0))],
    out_specs=[pl.BlockSpec((1, _RB, _C, _W), lambda r: (0, r, 0, 0))],
    out_shape=[jax.ShapeDtypeStruct((_B, _H, _C, _W), jnp.float32)],
)


def _post1_body(tab_ref, acc_ref, img_ref):
    del acc_ref
    y = tab_ref[:, :_C]
    img_ref[0] = jnp.transpose(y.reshape(_RB, _W, _C), (0, 2, 1))


_post1 = pl.pallas_call(
    _post1_body,
    grid=(_H // _RB,),
    in_specs=[pl.BlockSpec((_NROW, _CP), lambda r: (r, 0)),
              pl.BlockSpec(memory_space=pl.ANY)],
    out_specs=[pl.BlockSpec((1, _RB, _C, _W), lambda r: (1, r, 0, 0))],
    out_shape=[jax.ShapeDtypeStruct((_B, _H, _C, _W), jnp.float32)],
    input_output_aliases={1: 0},
)


def _warp_body(img_hbm, idxb_hbm, w00_hbm, w01_hbm, w10_hbm, w11_hbm,
               out_hbm, ib_v, idx_v, w_v, g_v, out_v, sem_in, sem_g, sem_out):
    wid = lax.axis_index("s") * _NC + lax.axis_index("c")
    w_hbms = (w00_hbm, w01_hbm, w10_hbm, w11_hbm)

    def chunk_of(i):
        return i * _NW + wid

    def in_copies(i, s):
        c = chunk_of(i)
        r2 = c // 2
        off = (c % 2) * _N
        cps = [pltpu.make_async_copy(
            idxb_hbm.at[r2, pl.ds(off, _N)], ib_v[s], sem_in[s])]
        for k in range(4):
            cps.append(pltpu.make_async_copy(
                w_hbms[k].at[r2, pl.ds(off, _N)], w_v[s].at[k], sem_in[s]))
        return cps

    def expand_idx(s):
        for g in range(_N // 16):
            rv = ib_v[s][pl.ds(g * 16, 16)]
            for c, delta in enumerate((0, 1, _W, _W + 1)):
                j = c * _N + g * 16
                idx_v[s][j // 128, pl.ds(j % 128, 16)] = rv + delta

    def gather_copies(i, s):
        return tuple(
            pltpu.make_async_copy(
                img_hbm.at[idx_v[s].at[q]],
                g_v[s].at[pl.ds(q * 128, 128)], sem_g[s])
            for q in range(2))

    def out_copy(i):
        c = chunk_of(i)
        return pltpu.make_async_copy(
            out_v, out_hbm.at[pl.ds(c * _N, _N)], sem_out)

    def fire(copies):
        for cp in copies:
            cp.start()

    def drain(copies):
        for cp in copies:
            cp.wait()

    def combine(i, s):
        gs = g_v[s]

        def grp_body(gq, _):
            b16 = gq * 16
            w0g = w_v[s][0, pl.ds(b16, 16)]
            w1g = w_v[s][1, pl.ds(b16, 16)]
            w2g = w_v[s][2, pl.ds(b16, 16)]
            w3g = w_v[s][3, pl.ds(b16, 16)]
            for li in range(16):
                p = b16 + li
                w0 = _lane_bcast(w0g, li)
                w1 = _lane_bcast(w1g, li)
                w2 = _lane_bcast(w2g, li)
                w3 = _lane_bcast(w3g, li)
                for blk in range(_NB):
                    csl = pl.ds(blk * 16, 16)
                    out_v[p, csl] = (
                        w0 * gs[0 * _N + p, csl] + w1 * gs[1 * _N + p, csl]
                        + w2 * gs[2 * _N + p, csl] + w3 * gs[3 * _N + p, csl])
            return 0

        lax.fori_loop(0, _N // 16, grp_body, 0)

    fire(in_copies(0, 0))
    fire(in_copies(1, 1))
    drain(in_copies(0, 0))
    expand_idx(0)
    fire(gather_copies(0, 0))

    def pair_body(k, _):
        for s in (0, 1):
            i = k * 2 + s

            @pl.when(i < _CPW - 1)
            def _():
                drain(in_copies(i + 1, 1 - s))
                expand_idx(1 - s)
                fire(gather_copies(i + 1, 1 - s))

            drain(gather_copies(i, s))

            @pl.when(i >= 1)
            def _():
                out_copy(i - 1).wait()

            combine(i, s)
            out_copy(i).start()

            @pl.when(i < _CPW - 2)
            def _():
                fire(in_copies(i + 2, s))
        return 0

    lax.fori_loop(0, _CPW // 2, pair_body, 0)
    out_copy(_CPW - 1).wait()


_GATHER_DNUMS = lax.GatherDimensionNumbers(
    offset_dims=(), collapsed_slice_dims=(0,), start_index_map=(0,))


def _lane_bcast(vec, lane):
    lidx = jnp.full((16, 1), lane, jnp.int32)
    return lax.gather(vec, lidx, _GATHER_DNUMS, (1,),
                      mode=lax.GatherScatterMode.PROMISE_IN_BOUNDS)


_warp = functools.partial(
    pl.kernel,
    out_type=jax.ShapeDtypeStruct((_HW, _CP), jnp.float32),
    mesh=plsc.VectorSubcoreMesh(core_axis_name="c", subcore_axis_name="s"),
    compiler_params=pltpu.CompilerParams(use_tc_tiling_on_sc=False),
    scratch_types=[
        [pltpu.VMEM((_N,), jnp.int32) for _ in range(2)],
        [pltpu.VMEM((2, 128), jnp.int32) for _ in range(2)],
        [pltpu.VMEM((4, _N), jnp.float32) for _ in range(2)],
        [pltpu.VMEM((4 * _N, _CP), jnp.float32)
         for _ in range(2)],
        pltpu.VMEM((_N, _CP), jnp.float32),
        [pltpu.SemaphoreType.DMA for _ in range(2)],
        [pltpu.SemaphoreType.DMA for _ in range(2)],
        pltpu.SemaphoreType.DMA,
    ],
)(_warp_body)


def kernel(img, trf):
    imgt = jnp.transpose(img, (0, 1, 3, 2))
    trft = jnp.transpose(trf, (0, 1, 3, 2))
    outps = []
    for b in range(_B):
        imgp, = _pre(imgt[b])
        idxb, w00, w01, w10, w11 = _prep(trft[b])
        outps.append(_warp(imgp, idxb, w00, w01, w10, w11))
    acc, = _post0(outps[0])
    outt, = _post1(outps[1], acc)
    return jnp.transpose(outt, (0, 1, 3, 2)), trf


# speedup vs baseline: 1.4839x; 1.1171x over previous
import functools

import jax
import jax.numpy as jnp
from jax import lax
from jax.experimental import pallas as pl
from jax.experimental.pallas import tpu as pltpu
from jax.experimental.pallas import tpu_sc as plsc

_B, _H, _W, _C = 2, 384, 384, 96
_HW = _H * _W
_CP = 128
_N = 64
_NC, _NS = 2, 16
_NW = _NC * _NS
_CHUNKS = _HW // _N
_CPW = _CHUNKS // _NW
_NB = _C // 16
_RB = 8
_NROW = _RB * _W
_IDXROWS = _HW // _CP


def _prep_body(trf_ref, idx_ref, w00_ref, w01_ref, w10_ref, w11_ref):
    r = pl.program_id(0)
    t = trf_ref[0]
    dx = t[:, 0, :]
    dy = t[:, 1, :]
    gi = ((lax.broadcasted_iota(jnp.int32, (_RB, _W), 0)
           + r * _RB).astype(jnp.float32))
    gj = lax.broadcasted_iota(jnp.int32, (_RB, _W), 1).astype(jnp.float32)
    cx = jnp.minimum(jnp.maximum(gi + dx, 0.0), float(_H - 1))
    cy = jnp.minimum(jnp.maximum(gj + dy, 0.0), float(_W - 1))
    xb = jnp.minimum(cx.astype(jnp.int32), _H - 2)
    yb = jnp.minimum(cy.astype(jnp.int32), _W - 2)
    fx = cx - xb.astype(jnp.float32)
    fy = cy - yb.astype(jnp.float32)
    rb = xb * _W + yb
    nch = _NROW // _CP
    idx_ref[...] = rb.reshape(nch, _CP)
    wxa = 1.0 - fx
    wya = 1.0 - fy
    w00_ref[...] = (wxa * wya).reshape(nch, _CP)
    w01_ref[...] = (wxa * fy).reshape(nch, _CP)
    w10_ref[...] = (fx * wya).reshape(nch, _CP)
    w11_ref[...] = (fx * fy).reshape(nch, _CP)


def _planar_spec():
    return pl.BlockSpec((_NROW // _CP, _CP), lambda r: (r, 0))


def _make_prep(b):
    return pl.pallas_call(
        _prep_body,
        grid=(_H // _RB,),
        in_specs=[pl.BlockSpec((1, _RB, 2, _W), lambda r: (b, r, 0, 0))],
        out_specs=[_planar_spec() for _ in range(5)],
        out_shape=[jax.ShapeDtypeStruct((_IDXROWS, _CP), jnp.int32)]
        + [jax.ShapeDtypeStruct((_IDXROWS, _CP), jnp.float32)
           for _ in range(4)],
    )


_preps = [_make_prep(b) for b in range(_B)]


def _pre_body(img_ref, tab_ref):
    x = img_ref[0]
    y = jnp.transpose(x, (0, 2, 1)).reshape(_NROW, _C)
    tab_ref[...] = jnp.concatenate(
        [y, jnp.zeros((_NROW, _CP - _C), jnp.float32)], axis=1)


def _make_pre(b):
    return pl.pallas_call(
        _pre_body,
        grid=(_H // _RB,),
        in_specs=[pl.BlockSpec((1, _RB, _C, _W), lambda r: (b, r, 0, 0))],
        out_specs=[pl.BlockSpec((_NROW, _CP), lambda r: (r, 0))],
        out_shape=[jax.ShapeDtypeStruct((_HW, _CP), jnp.float32)],
    )


_pres = [_make_pre(b) for b in range(_B)]


def _post0_body(tab_ref, img_ref):
    y = tab_ref[:, :_C]
    img_ref[0] = jnp.transpose(y.reshape(_RB, _W, _C), (0, 2, 1))


_post0 = pl.pallas_call(
    _post0_body,
    grid=(_H // _RB,),
    in_specs=[pl.BlockSpec((_NROW, _CP), lambda r: (r, 0))],
    out_specs=[pl.BlockSpec((1, _RB, _C, _W), lambda r: (0, r, 0, 0))],
    out_shape=[jax.ShapeDtypeStruct((_B, _H, _C, _W), jnp.float32)],
)


def _post1_body(tab_ref, acc_ref, img_ref):
    del acc_ref
    y = tab_ref[:, :_C]
    img_ref[0] = jnp.transpose(y.reshape(_RB, _W, _C), (0, 2, 1))


_post1 = pl.pallas_call(
    _post1_body,
    grid=(_H // _RB,),
    in_specs=[pl.BlockSpec((_NROW, _CP), lambda r: (r, 0)),
              pl.BlockSpec(memory_space=pl.ANY)],
    out_specs=[pl.BlockSpec((1, _RB, _C, _W), lambda r: (1, r, 0, 0))],
    out_shape=[jax.ShapeDtypeStruct((_B, _H, _C, _W), jnp.float32)],
    input_output_aliases={1: 0},
)


def _warp_body(img_hbm, idxb_hbm, w00_hbm, w01_hbm, w10_hbm, w11_hbm,
               out_hbm, ib_v, idx_v, w_v, g_v, out_v, sem_in, sem_g, sem_out):
    wid = lax.axis_index("s") * _NC + lax.axis_index("c")
    w_hbms = (w00_hbm, w01_hbm, w10_hbm, w11_hbm)

    def chunk_of(i):
        return i * _NW + wid

    def in_copies(i, s):
        c = chunk_of(i)
        r2 = c // 2
        off = (c % 2) * _N
        cps = [pltpu.make_async_copy(
            idxb_hbm.at[r2, pl.ds(off, _N)], ib_v[s], sem_in[s])]
        for k in range(4):
            cps.append(pltpu.make_async_copy(
                w_hbms[k].at[r2, pl.ds(off, _N)], w_v[s].at[k], sem_in[s]))
        return cps

    def expand_idx(s):
        for g in range(_N // 16):
            rv = ib_v[s][pl.ds(g * 16, 16)]
            for c, delta in enumerate((0, 1, _W, _W + 1)):
                j = c * _N + g * 16
                idx_v[s][j // 128, pl.ds(j % 128, 16)] = rv + delta

    def gather_copies(i, s):
        return tuple(
            pltpu.make_async_copy(
                img_hbm.at[idx_v[s].at[q]],
                g_v[s].at[pl.ds(q * 128, 128)], sem_g[s])
            for q in range(2))

    def out_copy(i):
        c = chunk_of(i)
        return pltpu.make_async_copy(
            out_v, out_hbm.at[pl.ds(c * _N, _N)], sem_out)

    def fire(copies):
        for cp in copies:
            cp.start()

    def drain(copies):
        for cp in copies:
            cp.wait()

    def combine(i, s):
        gs = g_v[s]

        def grp_body(gq, _):
            b16 = gq * 16
            w0g = w_v[s][0, pl.ds(b16, 16)]
            w1g = w_v[s][1, pl.ds(b16, 16)]
            w2g = w_v[s][2, pl.ds(b16, 16)]
            w3g = w_v[s][3, pl.ds(b16, 16)]
            for li in range(16):
                p = b16 + li
                w0 = _lane_bcast(w0g, li)
                w1 = _lane_bcast(w1g, li)
                w2 = _lane_bcast(w2g, li)
                w3 = _lane_bcast(w3g, li)
                for blk in range(_NB):
                    csl = pl.ds(blk * 16, 16)
                    out_v[p, csl] = (
                        w0 * gs[0 * _N + p, csl] + w1 * gs[1 * _N + p, csl]
                        + w2 * gs[2 * _N + p, csl] + w3 * gs[3 * _N + p, csl])
            return 0

        lax.fori_loop(0, _N // 16, grp_body, 0)

    fire(in_copies(0, 0))
    fire(in_copies(1, 1))
    drain(in_copies(0, 0))
    expand_idx(0)
    fire(gather_copies(0, 0))

    def pair_body(k, _):
        for s in (0, 1):
            i = k * 2 + s

            @pl.when(i < _CPW - 1)
            def _():
                drain(in_copies(i + 1, 1 - s))
                expand_idx(1 - s)
                fire(gather_copies(i + 1, 1 - s))

            drain(gather_copies(i, s))

            @pl.when(i >= 1)
            def _():
                out_copy(i - 1).wait()

            combine(i, s)
            out_copy(i).start()

            @pl.when(i < _CPW - 2)
            def _():
                fire(in_copies(i + 2, s))
        return 0

    lax.fori_loop(0, _CPW // 2, pair_body, 0)
    out_copy(_CPW - 1).wait()


_GATHER_DNUMS = lax.GatherDimensionNumbers(
    offset_dims=(), collapsed_slice_dims=(0,), start_index_map=(0,))


def _lane_bcast(vec, lane):
    lidx = jnp.full((16, 1), lane, jnp.int32)
    return lax.gather(vec, lidx, _GATHER_DNUMS, (1,),
                      mode=lax.GatherScatterMode.PROMISE_IN_BOUNDS)


_warp = functools.partial(
    pl.kernel,
    out_type=jax.ShapeDtypeStruct((_HW, _CP), jnp.float32),
    mesh=plsc.VectorSubcoreMesh(core_axis_name="c", subcore_axis_name="s"),
    compiler_params=pltpu.CompilerParams(use_tc_tiling_on_sc=False),
    scratch_types=[
        [pltpu.VMEM((_N,), jnp.int32) for _ in range(2)],
        [pltpu.VMEM((2, 128), jnp.int32) for _ in range(2)],
        [pltpu.VMEM((4, _N), jnp.float32) for _ in range(2)],
        [pltpu.VMEM((4 * _N, _CP), jnp.float32)
         for _ in range(2)],
        pltpu.VMEM((_N, _CP), jnp.float32),
        [pltpu.SemaphoreType.DMA for _ in range(2)],
        [pltpu.SemaphoreType.DMA for _ in range(2)],
        pltpu.SemaphoreType.DMA,
    ],
)(_warp_body)


def kernel(img, trf):
    imgt = jnp.transpose(img, (0, 1, 3, 2))
    trft = jnp.transpose(trf, (0, 1, 3, 2))
    outps = []
    for b in range(_B):
        imgp, = _pres[b](imgt)
        idxb, w00, w01, w10, w11 = _preps[b](trft)
        outps.append(_warp(imgp, idxb, w00, w01, w10, w11))
    acc, = _post0(outps[0])
    outt, = _post1(outps[1], acc)
    return jnp.transpose(outt, (0, 1, 3, 2)), trf
